# baseline (speedup 1.0000x reference)
import jax
import jax.numpy as jnp
from jax import lax
from jax.experimental import pallas as pl
from jax.experimental.pallas import tpu as pltpu
from jax.experimental.pallas import tpu_sc as plsc

N = 10000
E = 320000
C = 128
NPAD = 10240
NC, NS, L = 2, 16, 16
NW = NC * NS
EC = E // NW
K = 80
EPT = K * 128
BR = 16
NB = K // BR
RPT = NPAD // NS


def _rsqrt(x):
    i = plsc.bitcast(x, jnp.int32)
    i = jnp.int32(0x5F3759DF) - lax.shift_right_logical(i, 1)
    y = plsc.bitcast(i, jnp.float32)
    for _ in range(3):
        y = y * (1.5 - 0.5 * x * y * y)
    return y


def _sc_body(vx_h, vy_h, vz_h, src_h, dst_h, xw_h, out_h, w_hbm,
             dgo_sh, dgi_sh, out_sh, gsem, ssem0, ssem1):
    c = lax.axis_index("c")
    s = lax.axis_index("s")
    zv = jnp.zeros((L,), jnp.float32)
    own = c * NS + s

    _ns = jax.named_scope
    def phase0(zbd, zb2):
        @pl.loop(0, 640 // L)
        def _(i):
            zbd[pl.ds(i * L, L)] = zv

        @pl.loop(0, 40)
        def _(r):
            for k in range(C // L):
                zb2[r, pl.ds(k * L, L)] = zv

        pltpu.sync_copy(zbd, dgo_sh.at[pl.ds(640 * s, 640)])
        pltpu.sync_copy(zbd, dgi_sh.at[pl.ds(640 * s, 640)])
        for i in range(RPT // 40):
            pltpu.sync_copy(zb2, out_sh.at[pl.ds(RPT * s + 40 * i, 40)])

    with _ns("ph0_zero"):
        pl.run_scoped(phase0,
                      pltpu.VMEM((640,), jnp.float32),
                      pltpu.VMEM((40, C), jnp.float32))
        plsc.subcore_barrier()

    def phase1(vx_v, vy_v, vz_v, s16, d16, w16):
        pltpu.sync_copy(vx_h, vx_v)
        pltpu.sync_copy(vy_h, vy_v)
        pltpu.sync_copy(vz_h, vz_v)

        def edge_weights(chunk, keep_w):
            @pl.loop(0, NB)
            def _(b2):
                pltpu.sync_copy(src_h.at[chunk, pl.ds(b2 * BR, BR)], s16)
                pltpu.sync_copy(dst_h.at[chunk, pl.ds(b2 * BR, BR)], d16)

                @pl.loop(0, BR)
                def _(r):
                    for k in range(C // L):
                        sl = pl.ds(k * L, L)
                        si = s16[r, sl]
                        di = d16[r, sl]
                        dx = (plsc.load_gather(vx_v, [si])
                              - plsc.load_gather(vx_v, [di]))
                        dy = (plsc.load_gather(vy_v, [si])
                              - plsc.load_gather(vy_v, [di]))
                        dz = (plsc.load_gather(vz_v, [si])
                              - plsc.load_gather(vz_v, [di]))
                        ss = dx * dx + dy * dy + dz * dz + 1e-12
                        dist = ss * _rsqrt(ss)
                        w = jnp.exp(-dist)
                        eidx = (b2 * BR + r) * 128 + k * L + lax.iota(jnp.int32, L)
                        w = jnp.where(eidx < EC, w, 0.0)
                        w16[r, sl] = w
                    pltpu.sync_copy(w16.at[r], dgo_sh.at[s16.at[r]], add=True)
                    pltpu.sync_copy(w16.at[r], dgi_sh.at[d16.at[r]], add=True)

                if keep_w:
                    pltpu.sync_copy(w16, w_hbm.at[chunk, pl.ds(b2 * BR, BR)])

        edge_weights((1 - c) * NS + s, False)
        edge_weights(own, True)

    with _ns("ph1_weights_degs"):
        pl.run_scoped(phase1,
                      pltpu.VMEM((N,), jnp.float32),
                      pltpu.VMEM((N,), jnp.float32),
                      pltpu.VMEM((N,), jnp.float32),
                      pltpu.VMEM((BR, 128), jnp.int32),
                      pltpu.VMEM((BR, 128), jnp.int32),
                      pltpu.VMEM((BR, 128), jnp.float32))
        plsc.subcore_barrier()

    def phase15(dgo_v, dgi_v, s16, d16, w16):
        pltpu.sync_copy(dgo_sh, dgo_v)
        pltpu.sync_copy(dgi_sh, dgi_v)

        @pl.loop(0, NB)
        def _(b2):
            pltpu.sync_copy(src_h.at[own, pl.ds(b2 * BR, BR)], s16)
            pltpu.sync_copy(dst_h.at[own, pl.ds(b2 * BR, BR)], d16)
            pltpu.sync_copy(w_hbm.at[own, pl.ds(b2 * BR, BR)], w16)

            @pl.loop(0, BR)
            def _(r):
                for k in range(C // L):
                    sl = pl.ds(k * L, L)
                    p = (plsc.load_gather(dgo_v, [s16[r, sl]])
                         * plsc.load_gather(dgi_v, [d16[r, sl]]))
                    sq = p * _rsqrt(p)
                    w16[r, sl] = w16[r, sl] / (sq + 1e-8)

            pltpu.sync_copy(w16, w_hbm.at[own, pl.ds(b2 * BR, BR)])

    with _ns("ph15_norm"):
        pl.run_scoped(phase15,
                      pltpu.VMEM((NPAD,), jnp.float32),
                      pltpu.VMEM((NPAD,), jnp.float32),
                      pltpu.VMEM((BR, 128), jnp.int32),
                      pltpu.VMEM((BR, 128), jnp.int32),
                      pltpu.VMEM((BR, 128), jnp.float32))

    def phase2(rows0, rows1, di0, di1, si0, si1, sc0, sc1, wn0, wn1):
        rows = (rows0, rows1)
        di = (di0, di1)
        si = (si0, si1)
        sci = (sc0, sc1)
        wnb = (wn0, wn1)
        ssem = (ssem0, ssem1)

        pltpu.sync_copy(dst_h.at[own, 0], di0)
        pltpu.sync_copy(src_h.at[own, 0], si0)
        pltpu.sync_copy(w_hbm.at[own, 0], wn0)
        pltpu.async_copy(xw_h.at[di0], rows0, gsem)

        @pl.loop(0, K // 2)
        def _(t):
            for par in range(2):
                b = 2 * t + par

                pltpu.make_async_copy(
                    xw_h.at[di[par]], rows[par], gsem).wait()

                @pl.when(b >= 1)
                def _():
                    pltpu.make_async_copy(
                        rows[1 - par], out_sh.at[sci[1 - par]],
                        ssem[1 - par]).wait()

                @pl.when(b + 1 < K)
                def _():
                    pltpu.sync_copy(dst_h.at[own, b + 1], di[1 - par])
                    pltpu.sync_copy(src_h.at[own, b + 1], si[1 - par])
                    pltpu.sync_copy(w_hbm.at[own, b + 1], wnb[1 - par])
                    pltpu.async_copy(xw_h.at[di[1 - par]], rows[1 - par], gsem)

                for g in range(128 // L):
                    wn16 = wnb[par][pl.ds(g * L, L)]
                    for u in range(L):
                        wn = wn16[u]
                        row = g * L + u
                        for k in range(C // L):
                            sl = pl.ds(k * L, L)
                            rows[par][row, sl] = rows[par][row, sl] * wn

                for k in range(128 // L):
                    sl = pl.ds(k * L, L)
                    sci[par][sl] = si[par][sl]
                pltpu.async_copy(rows[par], out_sh.at[sci[par]],
                                 ssem[par], add=True)

        lastp = (K - 1) % 2
        pltpu.make_async_copy(rows[lastp], out_sh.at[sci[lastp]],
                              ssem[lastp]).wait()

    with _ns("ph2_message"):
        pl.run_scoped(phase2,
                      pltpu.VMEM((128, C), jnp.float32),
                      pltpu.VMEM((128, C), jnp.float32),
                      pltpu.VMEM((128,), jnp.int32),
                      pltpu.VMEM((128,), jnp.int32),
                      pltpu.VMEM((128,), jnp.int32),
                      pltpu.VMEM((128,), jnp.int32),
                      pltpu.VMEM((128,), jnp.int32),
                      pltpu.VMEM((128,), jnp.int32),
                      pltpu.VMEM((128,), jnp.float32),
                      pltpu.VMEM((128,), jnp.float32))
        plsc.subcore_barrier()

    with _ns("ph3_out"):
        pltpu.sync_copy(out_sh.at[pl.ds(RPT * s, RPT)],
                        out_h.at[c].at[pl.ds(RPT * s, RPT)])


def _mm_body(x_ref, w_ref, o_ref):
    o_ref[...] = jnp.dot(x_ref[...], w_ref[...],
                         preferred_element_type=jnp.float32)


def _comb_body(p_ref, b_ref, o_ref):
    o_ref[...] = p_ref[0] + p_ref[1] + b_ref[...]


@jax.jit
def kernel(features, vertices, edges, faces, W, b):
    del faces
    xw = pl.pallas_call(
        _mm_body,
        grid=(10,),
        in_specs=[pl.BlockSpec((N // 10, C), lambda i: (i, 0)),
                  pl.BlockSpec((C, C), lambda i: (0, 0))],
        out_specs=pl.BlockSpec((N // 10, C), lambda i: (i, 0)),
        out_shape=jax.ShapeDtypeStruct((N, C), jnp.float32),
    )(features, W)

    vx = vertices[:, 0]
    vy = vertices[:, 1]
    vz = vertices[:, 2]
    src = jnp.pad(edges[0].reshape(NW, EC), ((0, 0), (0, EPT - EC))
                  ).reshape(NW, K, 128).astype(jnp.int32)
    dst = jnp.pad(edges[1].reshape(NW, EC), ((0, 0), (0, EPT - EC))
                  ).reshape(NW, K, 128).astype(jnp.int32)

    mesh = plsc.VectorSubcoreMesh(core_axis_name="c", subcore_axis_name="s")
    partial, _ = pl.kernel(
        _sc_body,
        out_type=(jax.ShapeDtypeStruct((NC, NPAD, C), jnp.float32),
                  jax.ShapeDtypeStruct((NW, K, 128), jnp.float32)),
        mesh=mesh,
        compiler_params=pltpu.CompilerParams(needs_layout_passes=False),
        scratch_types=[pltpu.VMEM_SHARED((NPAD,), jnp.float32),
                       pltpu.VMEM_SHARED((NPAD,), jnp.float32),
                       pltpu.VMEM_SHARED((NPAD, C), jnp.float32),
                       pltpu.SemaphoreType.DMA,
                       pltpu.SemaphoreType.DMA,
                       pltpu.SemaphoreType.DMA],
    )(vx, vy, vz, src, dst, xw)

    out = pl.pallas_call(
        _comb_body,
        grid=(10,),
        in_specs=[pl.BlockSpec((NC, N // 10, C), lambda i: (0, i, 0)),
                  pl.BlockSpec((1, C), lambda i: (0, 0))],
        out_specs=pl.BlockSpec((N // 10, C), lambda i: (i, 0)),
        out_shape=jax.ShapeDtypeStruct((N, C), jnp.float32),
    )(partial, b.reshape(1, C))
    return out


# speedup vs baseline: 1.9931x; 1.9931x over previous
import jax
import jax.numpy as jnp
from jax import lax
from jax.experimental import pallas as pl
from jax.experimental.pallas import tpu as pltpu
from jax.experimental.pallas import tpu_sc as plsc

N = 10000
E = 320000
C = 128
NPAD = 10240
NC, NS, L = 2, 16, 16
NW = NC * NS
EC = E // NW
K = 80
EPT = K * 128
BR = 16
NB = K // BR
RPT = NPAD // NS


def _rsqrt(x):
    i = plsc.bitcast(x, jnp.int32)
    i = jnp.int32(0x5F3759DF) - lax.shift_right_logical(i, 1)
    y = plsc.bitcast(i, jnp.float32)
    for _ in range(3):
        y = y * (1.5 - 0.5 * x * y * y)
    return y


def _sc_body(vx_h, vy_h, vz_h, src_h, dst_h, xw_h, out_h, w_hbm,
             dgo_sh, dgi_sh, out_sh, gsem, ssem0, ssem1):
    c = lax.axis_index("c")
    s = lax.axis_index("s")
    zv = jnp.zeros((L,), jnp.float32)
    own = c * NS + s

    _ns = jax.named_scope
    def phase0(zbd, zb2):
        @pl.loop(0, 640 // L)
        def _(i):
            zbd[pl.ds(i * L, L)] = zv

        @pl.loop(0, 40)
        def _(r):
            for k in range(C // L):
                zb2[r, pl.ds(k * L, L)] = zv

        pltpu.sync_copy(zbd, dgo_sh.at[pl.ds(640 * s, 640)])
        pltpu.sync_copy(zbd, dgi_sh.at[pl.ds(640 * s, 640)])
        for i in range(RPT // 40):
            pltpu.sync_copy(zb2, out_sh.at[pl.ds(RPT * s + 40 * i, 40)])

    with _ns("ph0_zero"):
        pl.run_scoped(phase0,
                      pltpu.VMEM((640,), jnp.float32),
                      pltpu.VMEM((40, C), jnp.float32))
        plsc.subcore_barrier()

    def phase1(vx_v, vy_v, vz_v, s16, d16, w16):
        pltpu.sync_copy(vx_h, vx_v)
        pltpu.sync_copy(vy_h, vy_v)
        pltpu.sync_copy(vz_h, vz_v)

        def edge_weights(chunk, keep_w):
            @pl.loop(0, NB)
            def _(b2):
                pltpu.sync_copy(src_h.at[chunk, pl.ds(b2 * BR, BR)], s16)
                pltpu.sync_copy(dst_h.at[chunk, pl.ds(b2 * BR, BR)], d16)

                @pl.loop(0, BR)
                def _(r):
                    for k in range(C // L):
                        sl = pl.ds(k * L, L)
                        si = s16[r, sl]
                        di = d16[r, sl]
                        dx = (plsc.load_gather(vx_v, [si])
                              - plsc.load_gather(vx_v, [di]))
                        dy = (plsc.load_gather(vy_v, [si])
                              - plsc.load_gather(vy_v, [di]))
                        dz = (plsc.load_gather(vz_v, [si])
                              - plsc.load_gather(vz_v, [di]))
                        ss = dx * dx + dy * dy + dz * dz + 1e-12
                        dist = ss * _rsqrt(ss)
                        w = jnp.exp(-dist)
                        eidx = (b2 * BR + r) * 128 + k * L + lax.iota(jnp.int32, L)
                        w = jnp.where(eidx < EC, w, 0.0)
                        w16[r, sl] = w
                    pltpu.sync_copy(w16.at[r], dgo_sh.at[s16.at[r]], add=True)
                    pltpu.sync_copy(w16.at[r], dgi_sh.at[d16.at[r]], add=True)

                if keep_w:
                    pltpu.sync_copy(w16, w_hbm.at[chunk, pl.ds(b2 * BR, BR)])

        edge_weights((1 - c) * NS + s, False)
        edge_weights(own, True)

    with _ns("ph1_weights_degs"):
        pl.run_scoped(phase1,
                      pltpu.VMEM((N,), jnp.float32),
                      pltpu.VMEM((N,), jnp.float32),
                      pltpu.VMEM((N,), jnp.float32),
                      pltpu.VMEM((BR, 128), jnp.int32),
                      pltpu.VMEM((BR, 128), jnp.int32),
                      pltpu.VMEM((BR, 128), jnp.float32))
        plsc.subcore_barrier()

    def phase15(dgo_v, dgi_v, s16, d16, w16):
        pltpu.sync_copy(dgo_sh, dgo_v)
        pltpu.sync_copy(dgi_sh, dgi_v)

        @pl.loop(0, NB)
        def _(b2):
            pltpu.sync_copy(src_h.at[own, pl.ds(b2 * BR, BR)], s16)
            pltpu.sync_copy(dst_h.at[own, pl.ds(b2 * BR, BR)], d16)
            pltpu.sync_copy(w_hbm.at[own, pl.ds(b2 * BR, BR)], w16)

            @pl.loop(0, BR)
            def _(r):
                for k in range(C // L):
                    sl = pl.ds(k * L, L)
                    p = (plsc.load_gather(dgo_v, [s16[r, sl]])
                         * plsc.load_gather(dgi_v, [d16[r, sl]]))
                    sq = p * _rsqrt(p)
                    w16[r, sl] = w16[r, sl] / (sq + 1e-8)

            pltpu.sync_copy(w16, w_hbm.at[own, pl.ds(b2 * BR, BR)])

    with _ns("ph15_norm"):
        pl.run_scoped(phase15,
                      pltpu.VMEM((NPAD,), jnp.float32),
                      pltpu.VMEM((NPAD,), jnp.float32),
                      pltpu.VMEM((BR, 128), jnp.int32),
                      pltpu.VMEM((BR, 128), jnp.int32),
                      pltpu.VMEM((BR, 128), jnp.float32))

    SB = 16

    def phase2(rows0, rows1, sc0, sc1, sblk, dblk, wblk):
        rows = (rows0, rows1)
        sci = (sc0, sc1)
        ssem = (ssem0, ssem1)

        @pl.loop(0, K // SB)
        def _(sb):
            pltpu.sync_copy(src_h.at[own, pl.ds(sb * SB, SB)], sblk)
            pltpu.sync_copy(dst_h.at[own, pl.ds(sb * SB, SB)], dblk)
            pltpu.sync_copy(w_hbm.at[own, pl.ds(sb * SB, SB)], wblk)
            pltpu.async_copy(xw_h.at[dblk.at[0]], rows0, gsem)

            @pl.loop(0, SB // 2)
            def _(t):
                for par in range(2):
                    bb = 2 * t + par

                    pltpu.make_async_copy(
                        xw_h.at[dblk.at[bb]], rows[par], gsem).wait()

                    @pl.when(bb >= 1)
                    def _():
                        pltpu.make_async_copy(
                            rows[1 - par], out_sh.at[sci[1 - par]],
                            ssem[1 - par]).wait()

                    @pl.when(bb + 1 < SB)
                    def _():
                        pltpu.async_copy(xw_h.at[dblk.at[bb + 1]],
                                         rows[1 - par], gsem)

                    for g in range(128 // L):
                        wn16 = wblk[bb, pl.ds(g * L, L)]
                        for u in range(L):
                            wn = wn16[u]
                            row = g * L + u
                            for k in range(C // L):
                                sl = pl.ds(k * L, L)
                                rows[par][row, sl] = rows[par][row, sl] * wn

                    for k in range(128 // L):
                        sl = pl.ds(k * L, L)
                        sci[par][sl] = sblk[bb, sl]
                    pltpu.async_copy(rows[par], out_sh.at[sci[par]],
                                     ssem[par], add=True)

            pltpu.make_async_copy(rows[1], out_sh.at[sci[1]],
                                  ssem[1]).wait()

    with _ns("ph2_message"):
        pl.run_scoped(phase2,
                      pltpu.VMEM((128, C), jnp.float32),
                      pltpu.VMEM((128, C), jnp.float32),
                      pltpu.VMEM((128,), jnp.int32),
                      pltpu.VMEM((128,), jnp.int32),
                      pltpu.VMEM((SB, 128), jnp.int32),
                      pltpu.VMEM((SB, 128), jnp.int32),
                      pltpu.VMEM((SB, 128), jnp.float32))
        plsc.subcore_barrier()

    with _ns("ph3_out"):
        pltpu.sync_copy(out_sh.at[pl.ds(RPT * s, RPT)],
                        out_h.at[c].at[pl.ds(RPT * s, RPT)])


def _mm_body(x_ref, w_ref, o_ref):
    o_ref[...] = jnp.dot(x_ref[...], w_ref[...],
                         preferred_element_type=jnp.float32)


def _comb_body(p_ref, b_ref, o_ref):
    o_ref[...] = p_ref[0] + p_ref[1] + b_ref[...]


@jax.jit
def kernel(features, vertices, edges, faces, W, b):
    del faces
    xw = pl.pallas_call(
        _mm_body,
        grid=(10,),
        in_specs=[pl.BlockSpec((N // 10, C), lambda i: (i, 0)),
                  pl.BlockSpec((C, C), lambda i: (0, 0))],
        out_specs=pl.BlockSpec((N // 10, C), lambda i: (i, 0)),
        out_shape=jax.ShapeDtypeStruct((N, C), jnp.float32),
    )(features, W)

    vx = vertices[:, 0]
    vy = vertices[:, 1]
    vz = vertices[:, 2]
    padv = (jnp.arange(EPT - EC, dtype=jnp.int32) * 83) % N
    padv = jnp.broadcast_to(padv, (NW, EPT - EC))
    src = jnp.concatenate([edges[0].reshape(NW, EC).astype(jnp.int32), padv],
                          axis=1).reshape(NW, K, 128)
    dst = jnp.concatenate([edges[1].reshape(NW, EC).astype(jnp.int32), padv],
                          axis=1).reshape(NW, K, 128)

    mesh = plsc.VectorSubcoreMesh(core_axis_name="c", subcore_axis_name="s")
    partial, _ = pl.kernel(
        _sc_body,
        out_type=(jax.ShapeDtypeStruct((NC, NPAD, C), jnp.float32),
                  jax.ShapeDtypeStruct((NW, K, 128), jnp.float32)),
        mesh=mesh,
        compiler_params=pltpu.CompilerParams(needs_layout_passes=False),
        scratch_types=[pltpu.VMEM_SHARED((NPAD,), jnp.float32),
                       pltpu.VMEM_SHARED((NPAD,), jnp.float32),
                       pltpu.VMEM_SHARED((NPAD, C), jnp.float32),
                       pltpu.SemaphoreType.DMA,
                       pltpu.SemaphoreType.DMA,
                       pltpu.SemaphoreType.DMA],
    )(vx, vy, vz, src, dst, xw)

    out = pl.pallas_call(
        _comb_body,
        grid=(10,),
        in_specs=[pl.BlockSpec((NC, N // 10, C), lambda i: (0, i, 0)),
                  pl.BlockSpec((1, C), lambda i: (0, 0))],
        out_specs=pl.BlockSpec((N // 10, C), lambda i: (i, 0)),
        out_shape=jax.ShapeDtypeStruct((N, C), jnp.float32),
    )(partial, b.reshape(1, C))
    return out


# speedup vs baseline: 2.8270x; 1.4184x over previous
import jax
import jax.numpy as jnp
from jax import lax
from jax.experimental import pallas as pl
from jax.experimental.pallas import tpu as pltpu
from jax.experimental.pallas import tpu_sc as plsc

N = 10000
E = 320000
C = 128
NPAD = 10240
NC, NS, L = 2, 16, 16
NW = NC * NS
EC = E // NW
K = 80
EPT = K * 128
BR = 16
NB = K // BR
RPT = NPAD // NS


def _rsqrt(x):
    i = plsc.bitcast(x, jnp.int32)
    i = jnp.int32(0x5F3759DF) - lax.shift_right_logical(i, 1)
    y = plsc.bitcast(i, jnp.float32)
    for _ in range(3):
        y = y * (1.5 - 0.5 * x * y * y)
    return y


def _sc_body(vx_h, vy_h, vz_h, src_h, dst_h, xw_h, out_h, w_hbm,
             dgo_sh, dgi_sh, out_sh, gsem, ssem0, ssem1, ssem2, ssem3):
    c = lax.axis_index("c")
    s = lax.axis_index("s")
    zv = jnp.zeros((L,), jnp.float32)
    own = c * NS + s

    _ns = jax.named_scope
    def phase0(zbd, zb2):
        @pl.loop(0, 640 // L)
        def _(i):
            zbd[pl.ds(i * L, L)] = zv

        @pl.loop(0, 40)
        def _(r):
            for k in range(C // L):
                zb2[r, pl.ds(k * L, L)] = zv

        pltpu.sync_copy(zbd, dgo_sh.at[pl.ds(640 * s, 640)])
        pltpu.sync_copy(zbd, dgi_sh.at[pl.ds(640 * s, 640)])
        for i in range(RPT // 40):
            pltpu.sync_copy(zb2, out_sh.at[pl.ds(RPT * s + 40 * i, 40)])

    with _ns("ph0_zero"):
        pl.run_scoped(phase0,
                      pltpu.VMEM((640,), jnp.float32),
                      pltpu.VMEM((40, C), jnp.float32))
        plsc.subcore_barrier()

    def phase1(vx_v, vy_v, vz_v, s16, d16, w16):
        pltpu.sync_copy(vx_h, vx_v)
        pltpu.sync_copy(vy_h, vy_v)
        pltpu.sync_copy(vz_h, vz_v)

        def edge_weights(chunk, keep_w):
            @pl.loop(0, NB)
            def _(b2):
                pltpu.sync_copy(src_h.at[chunk, pl.ds(b2 * BR, BR)], s16)
                pltpu.sync_copy(dst_h.at[chunk, pl.ds(b2 * BR, BR)], d16)

                @pl.loop(0, BR)
                def _(r):
                    for k in range(C // L):
                        sl = pl.ds(k * L, L)
                        si = s16[r, sl]
                        di = d16[r, sl]
                        dx = (plsc.load_gather(vx_v, [si])
                              - plsc.load_gather(vx_v, [di]))
                        dy = (plsc.load_gather(vy_v, [si])
                              - plsc.load_gather(vy_v, [di]))
                        dz = (plsc.load_gather(vz_v, [si])
                              - plsc.load_gather(vz_v, [di]))
                        ss = dx * dx + dy * dy + dz * dz + 1e-12
                        dist = ss * _rsqrt(ss)
                        w = jnp.exp(-dist)
                        eidx = (b2 * BR + r) * 128 + k * L + lax.iota(jnp.int32, L)
                        w = jnp.where(eidx < EC, w, 0.0)
                        w16[r, sl] = w
                    pltpu.async_copy(w16.at[r], dgo_sh.at[s16.at[r]], gsem,
                                     add=True)
                    pltpu.async_copy(w16.at[r], dgi_sh.at[d16.at[r]], gsem,
                                     add=True)

                if keep_w:
                    pltpu.sync_copy(w16, w_hbm.at[chunk, pl.ds(b2 * BR, BR)])

                @pl.loop(0, BR)
                def _(r):
                    pltpu.make_async_copy(w16.at[r], dgo_sh.at[s16.at[r]],
                                          gsem).wait()
                    pltpu.make_async_copy(w16.at[r], dgi_sh.at[d16.at[r]],
                                          gsem).wait()

        edge_weights((1 - c) * NS + s, False)
        edge_weights(own, True)

    with _ns("ph1_weights_degs"):
        pl.run_scoped(phase1,
                      pltpu.VMEM((N,), jnp.float32),
                      pltpu.VMEM((N,), jnp.float32),
                      pltpu.VMEM((N,), jnp.float32),
                      pltpu.VMEM((BR, 128), jnp.int32),
                      pltpu.VMEM((BR, 128), jnp.int32),
                      pltpu.VMEM((BR, 128), jnp.float32))
        plsc.subcore_barrier()

    def phase15(dgo_v, dgi_v, s16, d16, w16):
        pltpu.sync_copy(dgo_sh, dgo_v)
        pltpu.sync_copy(dgi_sh, dgi_v)

        @pl.loop(0, NB)
        def _(b2):
            pltpu.sync_copy(src_h.at[own, pl.ds(b2 * BR, BR)], s16)
            pltpu.sync_copy(dst_h.at[own, pl.ds(b2 * BR, BR)], d16)
            pltpu.sync_copy(w_hbm.at[own, pl.ds(b2 * BR, BR)], w16)

            @pl.loop(0, BR)
            def _(r):
                for k in range(C // L):
                    sl = pl.ds(k * L, L)
                    p = (plsc.load_gather(dgo_v, [s16[r, sl]])
                         * plsc.load_gather(dgi_v, [d16[r, sl]]))
                    sq = p * _rsqrt(p)
                    w16[r, sl] = w16[r, sl] / (sq + 1e-8)

            pltpu.sync_copy(w16, w_hbm.at[own, pl.ds(b2 * BR, BR)])

    with _ns("ph15_norm"):
        pl.run_scoped(phase15,
                      pltpu.VMEM((NPAD,), jnp.float32),
                      pltpu.VMEM((NPAD,), jnp.float32),
                      pltpu.VMEM((BR, 128), jnp.int32),
                      pltpu.VMEM((BR, 128), jnp.int32),
                      pltpu.VMEM((BR, 128), jnp.float32))

    SB = 16
    UR = 64
    NU = SB * 128 // UR

    def phase2(rows0, rows1, rows2, rows3, sc0, sc1, sc2, sc3,
               sblk, dblk, wblk):
        rows = (rows0, rows1, rows2, rows3)
        sci = (sc0, sc1, sc2, sc3)
        ssem = (ssem0, ssem1, ssem2, ssem3)

        def didx(u, par):
            return dblk.at[lax.div(u, 2), pl.ds((par % 2) * UR, UR)]

        @pl.loop(0, K // SB)
        def _(sb):
            pltpu.sync_copy(src_h.at[own, pl.ds(sb * SB, SB)], sblk)
            pltpu.sync_copy(dst_h.at[own, pl.ds(sb * SB, SB)], dblk)
            pltpu.sync_copy(w_hbm.at[own, pl.ds(sb * SB, SB)], wblk)
            for p in range(3):
                pltpu.async_copy(xw_h.at[didx(jnp.int32(p), p)],
                                 rows[p], gsem)

            @pl.loop(0, NU // 4)
            def _(t):
                for par in range(4):
                    u = 4 * t + par
                    bb = lax.div(u, 2)
                    h = par % 2

                    pltpu.make_async_copy(
                        xw_h.at[didx(u, par)], rows[par], gsem).wait()

                    pp = (par + 3) % 4

                    @pl.when(u >= 1)
                    def _():
                        pltpu.make_async_copy(
                            rows[pp], out_sh.at[sci[pp]], ssem[pp]).wait()

                    @pl.when(u + 3 < NU)
                    def _():
                        pltpu.async_copy(xw_h.at[didx(u + 3, par + 3)],
                                         rows[pp], gsem)

                    @pl.loop(0, UR // L)
                    def _(g):
                        wn16 = wblk[bb, pl.ds(h * UR + g * L, L)]
                        for v in range(L):
                            wn = wn16[v]
                            for k in range(C // L):
                                sl = pl.ds(k * L, L)
                                row = g * L + v
                                rows[par][row, sl] = rows[par][row, sl] * wn

                    for k in range(UR // L):
                        sl = pl.ds(k * L, L)
                        sci[par][sl] = sblk[bb, pl.ds(h * UR + k * L, L)]
                    pltpu.async_copy(rows[par], out_sh.at[sci[par]],
                                     ssem[par], add=True)

            pltpu.make_async_copy(rows[3], out_sh.at[sci[3]],
                                  ssem[3]).wait()

    with _ns("ph2_message"):
        pl.run_scoped(phase2,
                      pltpu.VMEM((UR, C), jnp.float32),
                      pltpu.VMEM((UR, C), jnp.float32),
                      pltpu.VMEM((UR, C), jnp.float32),
                      pltpu.VMEM((UR, C), jnp.float32),
                      pltpu.VMEM((UR,), jnp.int32),
                      pltpu.VMEM((UR,), jnp.int32),
                      pltpu.VMEM((UR,), jnp.int32),
                      pltpu.VMEM((UR,), jnp.int32),
                      pltpu.VMEM((SB, 128), jnp.int32),
                      pltpu.VMEM((SB, 128), jnp.int32),
                      pltpu.VMEM((SB, 128), jnp.float32))
        plsc.subcore_barrier()

    with _ns("ph3_out"):
        pltpu.sync_copy(out_sh.at[pl.ds(RPT * s, RPT)],
                        out_h.at[c].at[pl.ds(RPT * s, RPT)])


def _mm_body(x_ref, w_ref, o_ref):
    o_ref[...] = jnp.dot(x_ref[...], w_ref[...],
                         preferred_element_type=jnp.float32)


def _comb_body(p_ref, b_ref, o_ref):
    o_ref[...] = p_ref[0] + p_ref[1] + b_ref[...]


@jax.jit
def kernel(features, vertices, edges, faces, W, b):
    del faces
    xw = pl.pallas_call(
        _mm_body,
        grid=(10,),
        in_specs=[pl.BlockSpec((N // 10, C), lambda i: (i, 0)),
                  pl.BlockSpec((C, C), lambda i: (0, 0))],
        out_specs=pl.BlockSpec((N // 10, C), lambda i: (i, 0)),
        out_shape=jax.ShapeDtypeStruct((N, C), jnp.float32),
    )(features, W)

    vx = vertices[:, 0]
    vy = vertices[:, 1]
    vz = vertices[:, 2]
    padv = (jnp.arange(EPT - EC, dtype=jnp.int32) * 83) % N
    padv = jnp.broadcast_to(padv, (NW, EPT - EC))
    src = jnp.concatenate([edges[0].reshape(NW, EC).astype(jnp.int32), padv],
                          axis=1).reshape(NW, K, 128)
    dst = jnp.concatenate([edges[1].reshape(NW, EC).astype(jnp.int32), padv],
                          axis=1).reshape(NW, K, 128)

    mesh = plsc.VectorSubcoreMesh(core_axis_name="c", subcore_axis_name="s")
    partial, _ = pl.kernel(
        _sc_body,
        out_type=(jax.ShapeDtypeStruct((NC, NPAD, C), jnp.float32),
                  jax.ShapeDtypeStruct((NW, K, 128), jnp.float32)),
        mesh=mesh,
        compiler_params=pltpu.CompilerParams(needs_layout_passes=False),
        scratch_types=[pltpu.VMEM_SHARED((NPAD,), jnp.float32),
                       pltpu.VMEM_SHARED((NPAD,), jnp.float32),
                       pltpu.VMEM_SHARED((NPAD, C), jnp.float32),
                       pltpu.SemaphoreType.DMA,
                       pltpu.SemaphoreType.DMA,
                       pltpu.SemaphoreType.DMA,
                       pltpu.SemaphoreType.DMA,
                       pltpu.SemaphoreType.DMA],
    )(vx, vy, vz, src, dst, xw)

    out = pl.pallas_call(
        _comb_body,
        grid=(10,),
        in_specs=[pl.BlockSpec((NC, N // 10, C), lambda i: (0, i, 0)),
                  pl.BlockSpec((1, C), lambda i: (0, 0))],
        out_specs=pl.BlockSpec((N // 10, C), lambda i: (i, 0)),
        out_shape=jax.ShapeDtypeStruct((N, C), jnp.float32),
    )(partial, b.reshape(1, C))
    return out
